# Initial kernel scaffold; baseline (speedup 1.0000x reference)
import jax
import jax.numpy as jnp
from jax.experimental import pallas as pl


def kernel(all_ts, W, cumhisto, clustering_flag):
    raise NotImplementedError("write your pallas kernel here")


# speedup vs baseline: 2.9222x; 2.9222x over previous
import functools

import jax
import jax.numpy as jnp
from jax.experimental import pallas as pl
from jax.experimental.pallas import tpu as pltpu


def _prep_body(x_ref, w_ref, wp_ref, acc_ref):
    i = pl.program_id(0)
    blk = x_ref[...]
    psum = jnp.sum(blk * blk, axis=0, keepdims=True)

    @pl.when(i == 0)
    def _():
        acc_ref[...] = psum

    @pl.when(i > 0)
    def _():
        acc_ref[...] = acc_ref[...] + psum

    @pl.when(i == pl.num_programs(0) - 1)
    def _():
        w = w_ref[...]
        cinv = jax.lax.rsqrt(acc_ref[...])
        rinv = jax.lax.rsqrt(jnp.sum(w * w, axis=1, keepdims=True))
        wp_ref[...] = (w * cinv * rinv).astype(jnp.bfloat16)


def _main_body(x_ref, wp_ref, out_ref):
    x = x_ref[...].astype(jnp.bfloat16)
    beta = jax.lax.dot_general(
        x, wp_ref[...],
        dimension_numbers=(((1,), (1,)), ((), ())),
        preferred_element_type=jnp.float32,
    )
    m = jnp.max(beta, axis=1, keepdims=True)
    e = jnp.exp(beta - m)
    out_ref[...] = e / jnp.sum(e, axis=1, keepdims=True)


@functools.partial(jax.jit, static_argnames=("interpret",))
def _snn_softmax(all_ts, W, interpret=False):
    B, TS = all_ts.shape
    N = W.shape[0]

    prep_blk = 2048
    wp = pl.pallas_call(
        _prep_body,
        grid=(B // prep_blk,),
        in_specs=[
            pl.BlockSpec((prep_blk, TS), lambda i: (i, 0)),
            pl.BlockSpec((N, TS), lambda i: (0, 0)),
        ],
        out_specs=pl.BlockSpec((N, TS), lambda i: (0, 0)),
        out_shape=jax.ShapeDtypeStruct((N, TS), jnp.bfloat16),
        scratch_shapes=[pltpu.VMEM((1, TS), jnp.float32)],
        interpret=interpret,
    )(all_ts, W)

    bm = 512
    out = pl.pallas_call(
        _main_body,
        grid=(B // bm,),
        in_specs=[
            pl.BlockSpec((bm, TS), lambda i: (i, 0)),
            pl.BlockSpec((N, TS), lambda i: (0, 0)),
        ],
        out_specs=pl.BlockSpec((bm, N), lambda i: (i, 0)),
        out_shape=jax.ShapeDtypeStruct((B, N), jnp.float32),
        interpret=interpret,
    )(all_ts, wp)
    return out


def kernel(all_ts, W, cumhisto, clustering_flag):
    x = all_ts.reshape(all_ts.shape[0], -1)
    return _snn_softmax(x, W)


# speedup vs baseline: 3.8386x; 1.3136x over previous
import functools

import jax
import jax.numpy as jnp
from jax.experimental import pallas as pl
from jax.experimental.pallas import tpu as pltpu


def _prep_body(x_ref, w_ref, wp_ref, acc_ref):
    i = pl.program_id(0)
    blk = x_ref[...]
    psum = jnp.sum(blk * blk, axis=0, keepdims=True)

    @pl.when(i == 0)
    def _():
        acc_ref[...] = psum

    @pl.when(i > 0)
    def _():
        acc_ref[...] = acc_ref[...] + psum

    @pl.when(i == pl.num_programs(0) - 1)
    def _():
        w = w_ref[...]
        cinv = jax.lax.rsqrt(acc_ref[...])
        rinv = jax.lax.rsqrt(jnp.sum(w * w, axis=1, keepdims=True))
        wp_ref[...] = (w * cinv * rinv).astype(jnp.bfloat16)


def _main_body(x_ref, wp_ref, out_ref):
    x = x_ref[...].astype(jnp.bfloat16)
    beta = jax.lax.dot_general(
        x, wp_ref[...],
        dimension_numbers=(((1,), (1,)), ((), ())),
        preferred_element_type=jnp.float32,
    )
    e = jnp.exp(beta)
    out_ref[...] = e * (1.0 / jnp.sum(e, axis=1, keepdims=True))


@functools.partial(jax.jit, static_argnames=("interpret",))
def _snn_softmax(all_ts, W, interpret=False):
    B, TS = all_ts.shape
    N = W.shape[0]

    prep_blk = 2048
    wp = pl.pallas_call(
        _prep_body,
        grid=(B // prep_blk,),
        in_specs=[
            pl.BlockSpec((prep_blk, TS), lambda i: (i, 0)),
            pl.BlockSpec((N, TS), lambda i: (0, 0)),
        ],
        out_specs=pl.BlockSpec((N, TS), lambda i: (0, 0)),
        out_shape=jax.ShapeDtypeStruct((N, TS), jnp.bfloat16),
        scratch_shapes=[pltpu.VMEM((1, TS), jnp.float32)],
        interpret=interpret,
    )(all_ts, W)

    bm = 1024
    out = pl.pallas_call(
        _main_body,
        grid=(B // bm,),
        in_specs=[
            pl.BlockSpec((bm, TS), lambda i: (i, 0)),
            pl.BlockSpec((N, TS), lambda i: (0, 0)),
        ],
        out_specs=pl.BlockSpec((bm, N), lambda i: (i, 0)),
        out_shape=jax.ShapeDtypeStruct((B, N), jnp.float32),
        interpret=interpret,
    )(all_ts, wp)
    return out


def kernel(all_ts, W, cumhisto, clustering_flag):
    x = all_ts.reshape(all_ts.shape[0], -1)
    return _snn_softmax(x, W)


# speedup vs baseline: 4.1868x; 1.0907x over previous
import functools

import jax
import jax.numpy as jnp
from jax.experimental import pallas as pl
from jax.experimental.pallas import tpu as pltpu


def _prep_body(x_ref, w_ref, wp_ref, acc_ref):
    i = pl.program_id(0)
    blk = x_ref[...]
    psum = jnp.sum(blk * blk, axis=0, keepdims=True)

    @pl.when(i == 0)
    def _():
        acc_ref[...] = psum

    @pl.when(i > 0)
    def _():
        acc_ref[...] = acc_ref[...] + psum

    @pl.when(i == pl.num_programs(0) - 1)
    def _():
        w = w_ref[...]
        cinv = jax.lax.rsqrt(acc_ref[...])
        rinv = jax.lax.rsqrt(jnp.sum(w * w, axis=1, keepdims=True))
        wp_ref[...] = (w * cinv * rinv).astype(jnp.bfloat16)


def _main_body(x_ref, wp_ref, out_ref):
    x = x_ref[...].astype(jnp.bfloat16)
    beta = jax.lax.dot_general(
        x, wp_ref[...],
        dimension_numbers=(((1,), (1,)), ((), ())),
        preferred_element_type=jnp.float32,
    )
    e = jnp.exp(beta)
    out_ref[...] = e * (1.0 / jnp.sum(e, axis=1, keepdims=True))


@functools.partial(jax.jit, static_argnames=("interpret",))
def _snn_softmax(all_ts, W, interpret=False):
    B, TS = all_ts.shape
    N = W.shape[0]

    prep_blk = 2048
    wp = pl.pallas_call(
        _prep_body,
        grid=(B // prep_blk,),
        in_specs=[
            pl.BlockSpec((prep_blk, TS), lambda i: (i, 0)),
            pl.BlockSpec((N, TS), lambda i: (0, 0)),
        ],
        out_specs=pl.BlockSpec((N, TS), lambda i: (0, 0)),
        out_shape=jax.ShapeDtypeStruct((N, TS), jnp.bfloat16),
        scratch_shapes=[pltpu.VMEM((1, TS), jnp.float32)],
        interpret=interpret,
    )(all_ts, W)

    bm = 2048
    out = pl.pallas_call(
        _main_body,
        grid=(B // bm,),
        in_specs=[
            pl.BlockSpec((bm, TS), lambda i: (i, 0)),
            pl.BlockSpec((N, TS), lambda i: (0, 0)),
        ],
        out_specs=pl.BlockSpec((bm, N), lambda i: (i, 0)),
        out_shape=jax.ShapeDtypeStruct((B, N), jnp.float32),
        interpret=interpret,
    )(all_ts, wp)
    return out


def kernel(all_ts, W, cumhisto, clustering_flag):
    x = all_ts.reshape(all_ts.shape[0], -1)
    return _snn_softmax(x, W)


# speedup vs baseline: 4.4336x; 1.0590x over previous
import functools

import jax
import jax.numpy as jnp
from jax.experimental import pallas as pl
from jax.experimental.pallas import tpu as pltpu


def _prep_body(x_ref, w_ref, wp_ref, acc_ref):
    i = pl.program_id(0)
    blk = x_ref[...]
    psum = jnp.sum(blk * blk, axis=0, keepdims=True)

    @pl.when(i == 0)
    def _():
        acc_ref[...] = psum

    @pl.when(i > 0)
    def _():
        acc_ref[...] = acc_ref[...] + psum

    @pl.when(i == pl.num_programs(0) - 1)
    def _():
        w = w_ref[...]
        cinv = jax.lax.rsqrt(acc_ref[...])
        rinv = jax.lax.rsqrt(jnp.sum(w * w, axis=1, keepdims=True))
        wp_ref[...] = (w * cinv * rinv).astype(jnp.bfloat16)


def _main_body(x_ref, wp_ref, out_ref):
    x = x_ref[...].astype(jnp.bfloat16)
    beta = jax.lax.dot_general(
        x, wp_ref[...],
        dimension_numbers=(((1,), (1,)), ((), ())),
        preferred_element_type=jnp.float32,
    )
    e = jnp.exp(beta)
    out_ref[...] = e * (1.0 / jnp.sum(e, axis=1, keepdims=True))


@functools.partial(jax.jit, static_argnames=("interpret",))
def _snn_softmax(all_ts, W, interpret=False):
    B, TS = all_ts.shape
    N = W.shape[0]

    prep_blk = 4096
    wp = pl.pallas_call(
        _prep_body,
        grid=(B // prep_blk,),
        in_specs=[
            pl.BlockSpec((prep_blk, TS), lambda i: (i, 0)),
            pl.BlockSpec((N, TS), lambda i: (0, 0)),
        ],
        out_specs=pl.BlockSpec((N, TS), lambda i: (0, 0)),
        out_shape=jax.ShapeDtypeStruct((N, TS), jnp.bfloat16),
        scratch_shapes=[pltpu.VMEM((1, TS), jnp.float32)],
        interpret=interpret,
    )(all_ts, W)

    bm = 4096
    out = pl.pallas_call(
        _main_body,
        grid=(B // bm,),
        in_specs=[
            pl.BlockSpec((bm, TS), lambda i: (i, 0)),
            pl.BlockSpec((N, TS), lambda i: (0, 0)),
        ],
        out_specs=pl.BlockSpec((bm, N), lambda i: (i, 0)),
        out_shape=jax.ShapeDtypeStruct((B, N), jnp.float32),
        interpret=interpret,
    )(all_ts, wp)
    return out


def kernel(all_ts, W, cumhisto, clustering_flag):
    x = all_ts.reshape(all_ts.shape[0], -1)
    return _snn_softmax(x, W)


# speedup vs baseline: 5.0035x; 1.1285x over previous
import functools

import jax
import jax.numpy as jnp
from jax.experimental import pallas as pl
from jax.experimental.pallas import tpu as pltpu

_BM = 2048


def _fused_body(x_ref, w_ref, out_ref, wp_ref):
    i = pl.program_id(0)

    @pl.when(i == 0)
    def _():
        x = x_ref[...]
        cinv = jax.lax.rsqrt(jnp.sum(x * x, axis=0, keepdims=True))
        w = w_ref[...]
        rinv = jax.lax.rsqrt(jnp.sum(w * w, axis=1, keepdims=True))
        wp_ref[...] = (w * cinv * rinv).astype(jnp.bfloat16)

    xblk = x_ref[pl.ds(i * _BM, _BM), :].astype(jnp.bfloat16)
    beta = jax.lax.dot_general(
        xblk, wp_ref[...],
        dimension_numbers=(((1,), (1,)), ((), ())),
        preferred_element_type=jnp.float32,
    )
    e = jnp.exp(beta)
    out_ref[...] = e * (1.0 / jnp.sum(e, axis=1, keepdims=True))


@functools.partial(jax.jit, static_argnames=("interpret",))
def _snn_softmax(all_ts, W, interpret=False):
    B, TS = all_ts.shape
    N = W.shape[0]
    out = pl.pallas_call(
        _fused_body,
        grid=(B // _BM,),
        in_specs=[
            pl.BlockSpec((B, TS), lambda i: (0, 0)),
            pl.BlockSpec((N, TS), lambda i: (0, 0)),
        ],
        out_specs=pl.BlockSpec((_BM, N), lambda i: (i, 0)),
        out_shape=jax.ShapeDtypeStruct((B, N), jnp.float32),
        scratch_shapes=[pltpu.VMEM((N, TS), jnp.bfloat16)],
        interpret=interpret,
    )(all_ts, W)
    return out


def kernel(all_ts, W, cumhisto, clustering_flag):
    x = all_ts.reshape(all_ts.shape[0], -1)
    return _snn_softmax(x, W)
